# baseline (speedup 1.0000x reference)
import jax
import jax.numpy as jnp
from jax.experimental import pallas as pl
from jax.experimental.pallas import tpu as pltpu

N_WAY, N_SUPPORT, EMB = 20, 5, 128
B, NBOX = 16, 8732


def _body(cls_t_ref, cls_p_ref, loc_p_ref, loc_t_ref, sup_ref, out_ref,
          acc_vec, acc_smem):
    i = pl.program_id(0)

    @pl.when(i == 0)
    def _init():
        acc_vec[...] = jnp.zeros_like(acc_vec)
        acc_smem[0] = 0.0
        acc_smem[1] = 0.0

    posf = (cls_t_ref[0] > 0).astype(jnp.float32)
    cls = cls_p_ref[0]
    acc_vec[...] += jax.lax.dot(posf, cls)

    diff = loc_p_ref[0] - loc_t_ref[0]
    a = jnp.abs(diff)
    sl1 = jnp.where(a < 1.0, 0.5 * diff * diff, a - 0.5)
    row = jnp.sum(sl1, axis=1, keepdims=True)
    acc_smem[0] += jnp.sum(jax.lax.dot(posf, row))
    acc_smem[1] += jnp.sum(posf)

    @pl.when(i == pl.num_programs(0) - 1)
    def _fin():
        num_pos = acc_smem[1]
        loc_loss = acc_smem[0]
        mean_q = acc_vec[...] / num_pos
        protos = (sup_ref[:, 0, :] + sup_ref[:, 1, :] + sup_ref[:, 2, :]
                  + sup_ref[:, 3, :] + sup_ref[:, 4, :]) * (1.0 / N_SUPPORT)
        d = jnp.sum((mean_q - protos) ** 2, axis=1)
        neg = -d
        m = jnp.max(neg)
        lse = m + jnp.log(jnp.sum(jnp.exp(neg - m)))
        cls_loss = lse - neg[0]
        out_ref[...] = jnp.full((1, 1), cls_loss + loc_loss / num_pos,
                                dtype=jnp.float32)


def kernel(loc_preds, loc_targets, cls_preds, cls_targets, supports):
    cls_t3 = cls_targets.reshape(B, 1, NBOX)
    out = pl.pallas_call(
        _body,
        grid=(B,),
        in_specs=[
            pl.BlockSpec((1, 1, NBOX), lambda i: (i, 0, 0)),
            pl.BlockSpec((1, NBOX, EMB), lambda i: (i, 0, 0)),
            pl.BlockSpec((1, NBOX, 4), lambda i: (i, 0, 0)),
            pl.BlockSpec((1, NBOX, 4), lambda i: (i, 0, 0)),
            pl.BlockSpec((N_WAY, N_SUPPORT, EMB), lambda i: (0, 0, 0)),
        ],
        out_specs=pl.BlockSpec((1, 1), lambda i: (0, 0)),
        out_shape=jax.ShapeDtypeStruct((1, 1), jnp.float32),
        scratch_shapes=[
            pltpu.VMEM((1, EMB), jnp.float32),
            pltpu.SMEM((2,), jnp.float32),
        ],
        compiler_params=pltpu.CompilerParams(
            dimension_semantics=("arbitrary",),
        ),
    )(cls_t3, cls_preds, loc_preds, loc_targets, supports)
    return out[0, 0]
